# baseline (speedup 1.0000x reference)
import jax
import jax.numpy as jnp
from jax.experimental import pallas as pl
from jax.experimental.pallas import tpu as pltpu

_NE = 1024
_ED = 256
_B = 32
_T = 1024
_N = _B * _T
_CC = 0.25

_BG = 4
_TC = 8
_TN = 8 * 128


def _vq_tile_kernel(x_ref, w_ref, w2_ref,
                    dist_ref, enc_ref, qt_ref, idx_ref, loss_ref, perp_ref,
                    hist_ref, msum_ref):
    g = pl.program_id(0)
    tc = pl.program_id(1)

    @pl.when((g == 0) & (tc == 0))
    def _init():
        hist_ref[...] = jnp.zeros_like(hist_ref)
        msum_ref[...] = jnp.zeros_like(msum_ref)

    xt = x_ref[...].reshape(_ED, _TN)
    w = w_ref[...]
    xw2 = jax.lax.dot_general(xt * (-2.0), w, (((0,), (1,)), ((), ())),
                              preferred_element_type=jnp.float32)
    x2 = jnp.sum(xt * xt, axis=0)[:, None]
    dist = (x2 + w2_ref[...]) + xw2
    dist_ref[...] = dist.reshape(1, 8, 1, _NE, 128)

    mn = jnp.min(dist, axis=1, keepdims=True)
    iota_l = jax.lax.broadcasted_iota(jnp.int32, (_TN, _NE), 1)
    idx = jnp.min(jnp.where(dist == mn, iota_l, _NE), axis=1).astype(jnp.int32)
    idx_ref[...] = idx.reshape(1, 8, 1, 1, 128)

    onehot = (iota_l == idx[:, None]).astype(jnp.float32)
    enc_ref[...] = onehot.reshape(1, 8, 1, _NE, 128)

    qt = jax.lax.dot_general(w, onehot, (((0,), (1,)), ((), ())),
                             preferred_element_type=jnp.float32)
    qt_ref[...] = qt.reshape(_ED, 1, 8, 128)

    hist_ref[...] += jnp.sum(onehot, axis=0, keepdims=True)
    msum_ref[...] += jnp.sum(mn).reshape(1, 1)

    @pl.when((g == _BG - 1) & (tc == _TC - 1))
    def _fin():
        avg = hist_ref[...] / _N
        ent = jnp.sum(avg * jnp.log(avg + 1e-10))
        perp_ref[...] = jnp.exp(-ent).reshape(1, 1)
        m = msum_ref[...] / (_N * _ED)
        loss_ref[...] = m + _CC * m


def kernel(inputs, W, compute_distances_if_possible):
    del compute_distances_if_possible
    x4 = inputs.reshape(_ED, _BG, 8, _T)
    w2 = jnp.sum(W ** 2, axis=1)[None, :]

    dist, enc, qt, idxo, loss, perp = pl.pallas_call(
        _vq_tile_kernel,
        grid=(_BG, _TC),
        in_specs=[
            pl.BlockSpec((_ED, 1, 8, 128), lambda g, t: (0, g, 0, t)),
            pl.BlockSpec((_NE, _ED), lambda g, t: (0, 0)),
            pl.BlockSpec((1, _NE), lambda g, t: (0, 0)),
        ],
        out_specs=[
            pl.BlockSpec((1, 8, 1, _NE, 128), lambda g, t: (g, 0, t, 0, 0)),
            pl.BlockSpec((1, 8, 1, _NE, 128), lambda g, t: (g, 0, t, 0, 0)),
            pl.BlockSpec((_ED, 1, 8, 128), lambda g, t: (0, g, 0, t)),
            pl.BlockSpec((1, 8, 1, 1, 128), lambda g, t: (g, 0, t, 0, 0)),
            pl.BlockSpec((1, 1), lambda g, t: (0, 0)),
            pl.BlockSpec((1, 1), lambda g, t: (0, 0)),
        ],
        out_shape=[
            jax.ShapeDtypeStruct((_BG, 8, _TC, _NE, 128), jnp.float32),
            jax.ShapeDtypeStruct((_BG, 8, _TC, _NE, 128), jnp.float32),
            jax.ShapeDtypeStruct((_ED, _BG, 8, _T), jnp.float32),
            jax.ShapeDtypeStruct((_BG, 8, _TC, 1, 128), jnp.int32),
            jax.ShapeDtypeStruct((1, 1), jnp.float32),
            jax.ShapeDtypeStruct((1, 1), jnp.float32),
        ],
        scratch_shapes=[
            pltpu.VMEM((1, _NE), jnp.float32),
            pltpu.VMEM((1, 1), jnp.float32),
        ],
    )(x4, W, w2)

    vq_loss = loss[0, 0]
    quantized_out = qt.reshape(_ED, _B, _T)
    perplexity = perp[0, 0]
    encodings_view = enc.reshape(_ED, _NE, 128)
    distances_view = dist.reshape(_ED, _NE, 128)
    return (vq_loss, quantized_out, perplexity, encodings_view,
            distances_view, idxo.reshape(_N, 1))


# speedup vs baseline: 1.1304x; 1.1304x over previous
import jax
import jax.numpy as jnp
from jax.experimental import pallas as pl
from jax.experimental.pallas import tpu as pltpu

_NE = 1024
_ED = 256
_B = 32
_T = 1024
_N = _B * _T
_CC = 0.25

_BG = 4
_TC = 8
_TN = 8 * 128


def _vq_tile_kernel(x_ref, w_ref, w2_ref,
                    dist_ref, enc_ref, qt_ref, idx_ref, loss_ref, perp_ref,
                    hist_ref, msum_ref):
    g = pl.program_id(0)
    tc = pl.program_id(1)

    @pl.when((g == 0) & (tc == 0))
    def _init():
        hist_ref[...] = jnp.zeros_like(hist_ref)
        msum_ref[...] = jnp.zeros_like(msum_ref)

    x = x_ref[...].reshape(_ED, _TN).T
    w = w_ref[...]
    xw2 = jax.lax.dot_general(x * (-2.0), w, (((1,), (1,)), ((), ())),
                              preferred_element_type=jnp.float32)
    x2 = jnp.sum(x * x, axis=1, keepdims=True)
    dist = (x2 + w2_ref[...]) + xw2
    dist_ref[...] = dist.reshape(1, 8, 1, _NE, 128)

    mn = jnp.min(dist, axis=1, keepdims=True)
    iota_l = jax.lax.broadcasted_iota(jnp.int32, (_TN, _NE), 1)
    idx = jnp.min(jnp.where(dist == mn, iota_l, _NE), axis=1).astype(jnp.int32)
    idx_ref[...] = idx.reshape(1, 8, 1, 1, 128)

    onehot = (iota_l == idx[:, None]).astype(jnp.float32)
    enc_ref[...] = onehot.reshape(1, 8, 1, _NE, 128)

    qt = jax.lax.dot_general(w, onehot, (((0,), (1,)), ((), ())),
                             preferred_element_type=jnp.float32)
    qt_ref[...] = qt.reshape(_ED, 1, 8, 128)

    ones_row = jnp.ones((8, _TN), jnp.float32)
    hist_inc = jax.lax.dot_general(ones_row, onehot, (((1,), (0,)), ((), ())),
                                   preferred_element_type=jnp.float32)
    hist_ref[...] += hist_inc[:1]
    msum_ref[...] += jnp.sum(mn).reshape(1, 1)

    @pl.when((g == _BG - 1) & (tc == _TC - 1))
    def _fin():
        avg = hist_ref[...] / _N
        ent = jnp.sum(avg * jnp.log(avg + 1e-10))
        perp_ref[...] = jnp.exp(-ent).reshape(1, 1)
        m = msum_ref[...] / (_N * _ED)
        loss_ref[...] = m + _CC * m


def kernel(inputs, W, compute_distances_if_possible):
    del compute_distances_if_possible
    x4 = inputs.reshape(_ED, _BG, 8, _T)
    w2 = jnp.sum(W ** 2, axis=1)[None, :]

    dist, enc, qt, idxo, loss, perp = pl.pallas_call(
        _vq_tile_kernel,
        grid=(_BG, _TC),
        in_specs=[
            pl.BlockSpec((_ED, 1, 8, 128), lambda g, t: (0, g, 0, t)),
            pl.BlockSpec((_NE, _ED), lambda g, t: (0, 0)),
            pl.BlockSpec((1, _NE), lambda g, t: (0, 0)),
        ],
        out_specs=[
            pl.BlockSpec((1, 8, 1, _NE, 128), lambda g, t: (g, 0, t, 0, 0)),
            pl.BlockSpec((1, 8, 1, _NE, 128), lambda g, t: (g, 0, t, 0, 0)),
            pl.BlockSpec((_ED, 1, 8, 128), lambda g, t: (0, g, 0, t)),
            pl.BlockSpec((1, 8, 1, 1, 128), lambda g, t: (g, 0, t, 0, 0)),
            pl.BlockSpec((1, 1), lambda g, t: (0, 0)),
            pl.BlockSpec((1, 1), lambda g, t: (0, 0)),
        ],
        out_shape=[
            jax.ShapeDtypeStruct((_BG, 8, _TC, _NE, 128), jnp.float32),
            jax.ShapeDtypeStruct((_BG, 8, _TC, _NE, 128), jnp.float32),
            jax.ShapeDtypeStruct((_ED, _BG, 8, _T), jnp.float32),
            jax.ShapeDtypeStruct((_BG, 8, _TC, 1, 128), jnp.int32),
            jax.ShapeDtypeStruct((1, 1), jnp.float32),
            jax.ShapeDtypeStruct((1, 1), jnp.float32),
        ],
        scratch_shapes=[
            pltpu.VMEM((1, _NE), jnp.float32),
            pltpu.VMEM((1, 1), jnp.float32),
        ],
    )(x4, W, w2)

    vq_loss = loss[0, 0]
    quantized_out = qt.reshape(_ED, _B, _T)
    perplexity = perp[0, 0]
    encodings_view = enc.reshape(_ED, _NE, 128)
    distances_view = dist.reshape(_ED, _NE, 128)
    return (vq_loss, quantized_out, perplexity, encodings_view,
            distances_view, idxo.reshape(_N, 1))


# speedup vs baseline: 1.3281x; 1.1749x over previous
import jax
import jax.numpy as jnp
from jax.experimental import pallas as pl
from jax.experimental.pallas import tpu as pltpu

_NE = 1024
_ED = 256
_B = 32
_T = 1024
_N = _B * _T
_CC = 0.25

_BG = 4
_TCH = 256
_TC = _T // _TCH
_VC = _TCH // 128
_TN = 8 * _TCH


def _vq_tile_kernel(x_ref, w_ref, w2_ref,
                    dist_ref, enc_ref, qt_ref, idx_ref, loss_ref, perp_ref,
                    hist_ref, msum_ref):
    g = pl.program_id(0)
    tc = pl.program_id(1)

    @pl.when((g == 0) & (tc == 0))
    def _init():
        hist_ref[...] = jnp.zeros_like(hist_ref)
        msum_ref[...] = jnp.zeros_like(msum_ref)

    x = x_ref[...].reshape(_ED, _TN).T
    w = w_ref[...]
    xw2 = jax.lax.dot_general(x * (-2.0), w, (((1,), (1,)), ((), ())),
                              preferred_element_type=jnp.float32)
    x2 = jnp.sum(x * x, axis=1, keepdims=True)
    dist = (x2 + w2_ref[...]) + xw2
    dist_ref[...] = dist.reshape(1, 8, _VC, _NE, 128)

    mn = jnp.min(dist, axis=1, keepdims=True)
    iota_l = jax.lax.broadcasted_iota(jnp.int32, (_TN, _NE), 1)
    idx = jnp.min(jnp.where(dist == mn, iota_l, _NE), axis=1).astype(jnp.int32)
    idx_ref[...] = idx.reshape(1, 8, _VC, 1, 128)

    onehot = (iota_l == idx[:, None]).astype(jnp.float32)
    enc_ref[...] = onehot.reshape(1, 8, _VC, _NE, 128)

    qt = jax.lax.dot_general(w, onehot, (((0,), (1,)), ((), ())),
                             preferred_element_type=jnp.float32)
    qt_ref[...] = qt.reshape(_ED, 1, 8, _TCH)

    hist_ref[...] += jnp.sum(onehot, axis=0, keepdims=True)
    msum_ref[...] += jnp.sum(mn).reshape(1, 1)

    @pl.when((g == _BG - 1) & (tc == _TC - 1))
    def _fin():
        avg = hist_ref[...] / _N
        ent = jnp.sum(avg * jnp.log(avg + 1e-10))
        perp_ref[...] = jnp.exp(-ent).reshape(1, 1)
        m = msum_ref[...] / (_N * _ED)
        loss_ref[...] = m + _CC * m


def kernel(inputs, W, compute_distances_if_possible):
    del compute_distances_if_possible
    x4 = inputs.reshape(_ED, _BG, 8, _T)
    w2 = jnp.sum(W ** 2, axis=1)[None, :]

    dist, enc, qt, idxo, loss, perp = pl.pallas_call(
        _vq_tile_kernel,
        grid=(_BG, _TC),
        in_specs=[
            pl.BlockSpec((_ED, 1, 8, _TCH), lambda g, t: (0, g, 0, t)),
            pl.BlockSpec((_NE, _ED), lambda g, t: (0, 0)),
            pl.BlockSpec((1, _NE), lambda g, t: (0, 0)),
        ],
        out_specs=[
            pl.BlockSpec((1, 8, _VC, _NE, 128), lambda g, t: (g, 0, t, 0, 0)),
            pl.BlockSpec((1, 8, _VC, _NE, 128), lambda g, t: (g, 0, t, 0, 0)),
            pl.BlockSpec((_ED, 1, 8, _TCH), lambda g, t: (0, g, 0, t)),
            pl.BlockSpec((1, 8, _VC, 1, 128), lambda g, t: (g, 0, t, 0, 0)),
            pl.BlockSpec((1, 1), lambda g, t: (0, 0)),
            pl.BlockSpec((1, 1), lambda g, t: (0, 0)),
        ],
        out_shape=[
            jax.ShapeDtypeStruct((_BG, 8, _TC * _VC, _NE, 128), jnp.float32),
            jax.ShapeDtypeStruct((_BG, 8, _TC * _VC, _NE, 128), jnp.float32),
            jax.ShapeDtypeStruct((_ED, _BG, 8, _T), jnp.float32),
            jax.ShapeDtypeStruct((_BG, 8, _TC * _VC, 1, 128), jnp.int32),
            jax.ShapeDtypeStruct((1, 1), jnp.float32),
            jax.ShapeDtypeStruct((1, 1), jnp.float32),
        ],
        scratch_shapes=[
            pltpu.VMEM((1, _NE), jnp.float32),
            pltpu.VMEM((1, 1), jnp.float32),
        ],
    )(x4, W, w2)

    vq_loss = loss[0, 0]
    quantized_out = qt.reshape(_ED, _B, _T)
    perplexity = perp[0, 0]
    encodings_view = enc.reshape(_ED, _NE, 128)
    distances_view = dist.reshape(_ED, _NE, 128)
    return (vq_loss, quantized_out, perplexity, encodings_view,
            distances_view, idxo.reshape(_N, 1))
